# baseline (speedup 1.0000x reference)
import jax
import jax.numpy as jnp
from jax.experimental import pallas as pl
from jax.experimental.pallas import tpu as pltpu

_BATCH = 1024
_N_CONT = 13
_N_CAT = 26
_VOCAB = 1000
_EMB = 50
_TOTAL_IN = _N_CONT + _N_CAT * _VOCAB
_TOTAL_OUT = _N_CONT + _N_CAT * _EMB
_TILE_B = 256
_SHIFT = _N_CONT % 8
_WPAD = _VOCAB + 8

_STARTS = [_N_CONT + f * _VOCAB for f in range(_N_CAT)]
_ALIGNED = [s - _SHIFT for s in _STARTS]


def _body(x_ref, w_ref, o_ref):
    o_ref[:_N_CONT, :] = x_ref[:_N_CONT, :]
    for f in range(_N_CAT):
        a = _ALIGNED[f]
        w = min(_WPAD, _TOTAL_IN - a)
        o_ref[_N_CONT + f * _EMB : _N_CONT + (f + 1) * _EMB, :] = jnp.dot(
            w_ref[f, :, :w],
            x_ref[a : a + w, :],
            preferred_element_type=jnp.float32,
            precision=jax.lax.Precision.DEFAULT,
        )


def kernel(inputs, embeddings):
    xt = inputs.T
    wt = embeddings.transpose(0, 2, 1)
    w2 = jnp.pad(wt, ((0, 0), (0, 0), (_SHIFT, _WPAD - _VOCAB - _SHIFT)))

    out_t = pl.pallas_call(
        _body,
        grid=(_BATCH // _TILE_B,),
        in_specs=[
            pl.BlockSpec((_TOTAL_IN, _TILE_B), lambda i: (0, i)),
            pl.BlockSpec((_N_CAT, _EMB, _WPAD), lambda i: (0, 0, 0)),
        ],
        out_specs=pl.BlockSpec((_TOTAL_OUT, _TILE_B), lambda i: (0, i)),
        out_shape=jax.ShapeDtypeStruct((_TOTAL_OUT, _BATCH), jnp.float32),
        compiler_params=pltpu.CompilerParams(
            vmem_limit_bytes=100 * 1024 * 1024,
        ),
    )(xt, w2)
    return out_t.T


# speedup vs baseline: 1.2241x; 1.2241x over previous
import jax
import jax.numpy as jnp
from jax.experimental import pallas as pl
from jax.experimental.pallas import tpu as pltpu

_BATCH = 1024
_N_CONT = 13
_N_CAT = 26
_VOCAB = 1000
_EMB = 50
_TOTAL_IN = _N_CONT + _N_CAT * _VOCAB
_TOTAL_OUT = _N_CONT + _N_CAT * _EMB
_TILE_B = 128
_STARTS = [_N_CONT + f * _VOCAB for f in range(_N_CAT)]


def _body(x_ref, w_ref, o_ref):
    o_ref[:_N_CONT, :] = x_ref[:_N_CONT, :]
    for f in range(_N_CAT):
        s = _STARTS[f]
        o_ref[_N_CONT + f * _EMB : _N_CONT + (f + 1) * _EMB, :] = jnp.dot(
            w_ref[f],
            x_ref[s : s + _VOCAB, :],
            preferred_element_type=jnp.float32,
            precision=jax.lax.Precision.DEFAULT,
        )


def kernel(inputs, embeddings):
    xt = inputs.T
    wt = embeddings.transpose(0, 2, 1)

    out_t = pl.pallas_call(
        _body,
        grid=(_BATCH // _TILE_B,),
        in_specs=[
            pl.BlockSpec((_TOTAL_IN, _TILE_B), lambda i: (0, i)),
            pl.BlockSpec((_N_CAT, _EMB, _VOCAB), lambda i: (0, 0, 0)),
        ],
        out_specs=pl.BlockSpec((_TOTAL_OUT, _TILE_B), lambda i: (0, i)),
        out_shape=jax.ShapeDtypeStruct((_TOTAL_OUT, _BATCH), jnp.float32),
    )(xt, wt)
    return out_t.T


# speedup vs baseline: 1.2252x; 1.0010x over previous
import jax
import jax.numpy as jnp
from jax.experimental import pallas as pl

_BATCH = 1024
_N_CONT = 13
_N_CAT = 26
_VOCAB = 1000
_EMB = 50
_TOTAL_IN = _N_CONT + _N_CAT * _VOCAB
_TOTAL_OUT = _N_CONT + _N_CAT * _EMB
_TILE_B = 128
_STARTS = [_N_CONT + f * _VOCAB for f in range(_N_CAT)]


def _body(x_ref, w_ref, o_ref):
    o_ref[:_N_CONT, :] = x_ref[:_N_CONT, :]
    for f in range(_N_CAT):
        s = _STARTS[f]
        o_ref[_N_CONT + f * _EMB : _N_CONT + (f + 1) * _EMB, :] = jnp.dot(
            w_ref[f],
            x_ref[s : s + _VOCAB, :],
            preferred_element_type=jnp.float32,
            precision=jax.lax.Precision.DEFAULT,
        )


def kernel(inputs, embeddings):
    xt = inputs.T
    wt = embeddings.transpose(0, 2, 1)

    out_t = pl.pallas_call(
        _body,
        grid=(_BATCH // _TILE_B,),
        in_specs=[
            pl.BlockSpec((_TOTAL_IN, _TILE_B), lambda i: (0, i)),
            pl.BlockSpec((_N_CAT, _EMB, _VOCAB), lambda i: (0, 0, 0)),
        ],
        out_specs=pl.BlockSpec((_TOTAL_OUT, _TILE_B), lambda i: (0, i)),
        out_shape=jax.ShapeDtypeStruct((_TOTAL_OUT, _BATCH), jnp.float32),
    )(xt, wt)
    return out_t.T
